# Initial kernel scaffold; baseline (speedup 1.0000x reference)
import jax
import jax.numpy as jnp
from jax.experimental import pallas as pl


def kernel(x, Wr, br, W1, b1, W2, b2):
    raise NotImplementedError("write your pallas kernel here")


# speedup vs baseline: 5.1516x; 5.1516x over previous
import functools

import jax
import jax.numpy as jnp
from jax import lax
from jax.experimental import pallas as pl
from jax.experimental.pallas import tpu as pltpu
from jax.experimental.pallas import tpu_sc as plsc

DM = 768
DFF = 3072
NE = 64
TOPK = 2
NT = 2048
NPAIR = NT * TOPK
TM = 128
TILES_MAX = (NPAIR + NE * TM) // TM
NMAX = TILES_MAX * TM
FF = 768
NFF = DFF // FF

NW = 32
_SC_MESH = dict(core_axis_name="c", subcore_axis_name="s")


def _router_body(x_ref, wr_ref, br_ref, idx_ref, gate_ref):
    x = x_ref[...]
    logits = jnp.dot(x, wr_ref[...], preferred_element_type=jnp.float32)
    logits = logits + br_ref[...]
    m = jnp.max(logits, axis=1, keepdims=True)
    ex = jnp.exp(logits - m)
    probs = ex / jnp.sum(ex, axis=1, keepdims=True)
    iota = lax.broadcasted_iota(jnp.int32, (NT, NE), 1)
    v1 = jnp.max(probs, axis=1, keepdims=True)
    i1 = jnp.min(jnp.where(probs == v1, iota, NE), axis=1, keepdims=True)
    masked = jnp.where(iota == i1, -1.0, probs)
    v2 = jnp.max(masked, axis=1, keepdims=True)
    i2 = jnp.min(jnp.where(masked == v2, iota, NE), axis=1, keepdims=True)
    s = v1 + v2
    idx_ref[...] = jnp.concatenate([i1, i2], axis=1)
    gate_ref[...] = jnp.concatenate([v1 / s, v2 / s], axis=1)


def _router(xf, Wr, br):
    return pl.pallas_call(
        _router_body,
        out_shape=(
            jax.ShapeDtypeStruct((NT, TOPK), jnp.int32),
            jax.ShapeDtypeStruct((NT, TOPK), jnp.float32),
        ),
    )(xf, Wr, br.reshape(1, NE))


def _plan(idx, gates):
    e_flat = idx.reshape(-1)
    g_flat = gates.reshape(-1)
    counts = jnp.zeros((NE,), jnp.int32).at[e_flat].add(1)
    tiles_e = (counts + TM - 1) // TM
    cum_tiles = jnp.cumsum(tiles_e)
    tile_off_e = cum_tiles - tiles_e
    total_tiles = cum_tiles[-1]
    off_e = jnp.cumsum(counts) - counts
    order = jnp.argsort(e_flat, stable=True).astype(jnp.int32)
    e_sorted = e_flat[order]
    s_idx = jnp.arange(NPAIR, dtype=jnp.int32)
    ps = tile_off_e[e_sorted] * TM + (s_idx - off_e[e_sorted])
    token_src = jnp.zeros((NMAX,), jnp.int32).at[ps].set(order // TOPK)
    gate_sorted = jnp.zeros((NMAX,), jnp.float32).at[ps].set(g_flat[order])
    pos = jnp.zeros((NPAIR,), jnp.int32).at[order].set(ps)
    t_idx = jnp.arange(TILES_MAX, dtype=jnp.int32)
    valid = t_idx < total_tiles
    expert_of_tile = jnp.searchsorted(cum_tiles, t_idx, side="right")
    tile_expert = jnp.where(valid, jnp.clip(expert_of_tile, 0, NE - 1),
                            e_sorted[-1]).astype(jnp.int32)
    row_block = jnp.where(valid, t_idx, total_tiles - 1).astype(jnp.int32)
    return (token_src, gate_sorted, pos[0::2], pos[1::2],
            tile_expert, row_block, valid.astype(jnp.int32))


_DISP_CHUNK = 128
_DISP_PER_W = NMAX // NW


def _dispatch_body(xf_hbm, idx_hbm, out_hbm, idx_v, rows_v, sem):
    wid = lax.axis_index("s") * 2 + lax.axis_index("c")
    for c in range(_DISP_PER_W // _DISP_CHUNK):
        base = wid * _DISP_PER_W + c * _DISP_CHUNK
        pltpu.sync_copy(idx_hbm.at[pl.ds(base, _DISP_CHUNK)], idx_v)
        pltpu.async_copy(xf_hbm.at[idx_v], rows_v, sem).wait()
        pltpu.sync_copy(rows_v, out_hbm.at[pl.ds(base, _DISP_CHUNK)])


def _dispatch(xf, token_src):
    k = functools.partial(
        pl.kernel,
        mesh=plsc.VectorSubcoreMesh(**_SC_MESH),
        out_type=jax.ShapeDtypeStruct((NMAX, DM), jnp.float32),
        scratch_types=[
            pltpu.VMEM((_DISP_CHUNK,), jnp.int32),
            pltpu.VMEM((_DISP_CHUNK, DM), jnp.float32),
            pltpu.SemaphoreType.DMA,
        ],
    )(_dispatch_body)
    return k(xf, token_src)


def _ffn_body(te_ref, rb_ref, va_ref, x_ref, w1_ref, b1_ref, w2_ref, b2_ref,
              g_ref, y_ref):
    t = pl.program_id(0)
    f = pl.program_id(1)

    @pl.when(va_ref[t] == 1)
    def _():
        x = x_ref[...]
        h = jnp.dot(x, w1_ref[0], preferred_element_type=jnp.float32)
        h = h + b1_ref[0]
        h = 0.5 * h * (1.0 + lax.erf(h * 0.7071067811865476))
        yp = jnp.dot(h, w2_ref[0], preferred_element_type=jnp.float32)

        @pl.when(f == 0)
        def _():
            y_ref[...] = yp

        @pl.when(f != 0)
        def _():
            y_ref[...] = y_ref[...] + yp

        @pl.when(f == NFF - 1)
        def _():
            y_ref[...] = (y_ref[...] + b2_ref[0]) * g_ref[...]


def _frozen_f(va_ref, t, f):
    return jnp.where(va_ref[t] == 1, f, NFF - 1)


def _ffn(X_sorted, W1, b1, W2, b2, gate_sorted, tile_expert, row_block, valid):
    grid_spec = pltpu.PrefetchScalarGridSpec(
        num_scalar_prefetch=3,
        grid=(TILES_MAX, NFF),
        in_specs=[
            pl.BlockSpec((TM, DM), lambda t, f, te, rb, va: (rb[t], 0)),
            pl.BlockSpec((1, DM, FF),
                         lambda t, f, te, rb, va: (te[t], 0, _frozen_f(va, t, f))),
            pl.BlockSpec((1, 1, FF),
                         lambda t, f, te, rb, va: (te[t], 0, _frozen_f(va, t, f))),
            pl.BlockSpec((1, FF, DM),
                         lambda t, f, te, rb, va: (te[t], _frozen_f(va, t, f), 0)),
            pl.BlockSpec((1, 1, DM), lambda t, f, te, rb, va: (te[t], 0, 0)),
            pl.BlockSpec((TM, 1), lambda t, f, te, rb, va: (rb[t], 0)),
        ],
        out_specs=pl.BlockSpec((TM, DM), lambda t, f, te, rb, va: (rb[t], 0)),
    )
    return pl.pallas_call(
        _ffn_body,
        grid_spec=grid_spec,
        out_shape=jax.ShapeDtypeStruct((NMAX, DM), jnp.float32),
    )(tile_expert, row_block, valid, X_sorted, W1, b1.reshape(NE, 1, DFF),
      W2, b2.reshape(NE, 1, DM), gate_sorted.reshape(NMAX, 1))


_COMB_PER_W = NT // NW


def _combine_body(y_hbm, p0_hbm, p1_hbm, out_hbm, i0_v, i1_v, r0_v, r1_v, sem):
    wid = lax.axis_index("s") * 2 + lax.axis_index("c")
    base = wid * _COMB_PER_W
    pltpu.sync_copy(p0_hbm.at[pl.ds(base, _COMB_PER_W)], i0_v)
    pltpu.sync_copy(p1_hbm.at[pl.ds(base, _COMB_PER_W)], i1_v)
    pltpu.async_copy(y_hbm.at[i0_v], r0_v, sem).wait()
    pltpu.async_copy(y_hbm.at[i1_v], r1_v, sem).wait()

    def row(rr, _):
        for cc in range(DM // 16):
            sl = pl.ds(cc * 16, 16)
            r0_v[rr, sl] = r0_v[rr, sl] + r1_v[rr, sl]
        return 0

    lax.fori_loop(0, _COMB_PER_W, row, 0)
    pltpu.sync_copy(r0_v, out_hbm.at[pl.ds(base, _COMB_PER_W)])


def _combine(Y, pos0, pos1):
    k = functools.partial(
        pl.kernel,
        mesh=plsc.VectorSubcoreMesh(**_SC_MESH),
        out_type=jax.ShapeDtypeStruct((NT, DM), jnp.float32),
        scratch_types=[
            pltpu.VMEM((_COMB_PER_W,), jnp.int32),
            pltpu.VMEM((_COMB_PER_W,), jnp.int32),
            pltpu.VMEM((_COMB_PER_W, DM), jnp.float32),
            pltpu.VMEM((_COMB_PER_W, DM), jnp.float32),
            pltpu.SemaphoreType.DMA,
        ],
    )(_combine_body)
    return k(Y, pos0, pos1)


def kernel(x, Wr, br, W1, b1, W2, b2):
    B, S, D = x.shape
    xf = x.reshape(-1, D)
    idx, gates = _router(xf, Wr, br)
    token_src, gate_sorted, pos0, pos1, tile_expert, row_block, valid = _plan(
        idx, gates)
    X_sorted = _dispatch(xf, token_src)
    Y = _ffn(X_sorted, W1, b1, W2, b2, gate_sorted, tile_expert, row_block,
             valid)
    out = _combine(Y, pos0, pos1)
    return out.reshape(B, S, D)


# speedup vs baseline: 5.5448x; 1.0763x over previous
import functools

import jax
import jax.numpy as jnp
from jax import lax
from jax.experimental import pallas as pl
from jax.experimental.pallas import tpu as pltpu
from jax.experimental.pallas import tpu_sc as plsc

DM = 768
DFF = 3072
NE = 64
TOPK = 2
NT = 2048
NPAIR = NT * TOPK
TM = 128
TILES_MAX = (NPAIR + NE * TM) // TM
NMAX = TILES_MAX * TM
FF = 3072
NFF = DFF // FF

NW = 32
_SC_MESH = dict(core_axis_name="c", subcore_axis_name="s")


def _router_body(x_ref, wr_ref, br_ref, idx_ref, gate_ref):
    x = x_ref[...]
    logits = jnp.dot(x, wr_ref[...], preferred_element_type=jnp.float32)
    logits = logits + br_ref[...]
    m = jnp.max(logits, axis=1, keepdims=True)
    ex = jnp.exp(logits - m)
    probs = ex / jnp.sum(ex, axis=1, keepdims=True)
    iota = lax.broadcasted_iota(jnp.int32, (NT, NE), 1)
    v1 = jnp.max(probs, axis=1, keepdims=True)
    i1 = jnp.min(jnp.where(probs == v1, iota, NE), axis=1, keepdims=True)
    masked = jnp.where(iota == i1, -1.0, probs)
    v2 = jnp.max(masked, axis=1, keepdims=True)
    i2 = jnp.min(jnp.where(masked == v2, iota, NE), axis=1, keepdims=True)
    s = v1 + v2
    idx_ref[...] = jnp.concatenate([i1, i2], axis=1)
    gate_ref[...] = jnp.concatenate([v1 / s, v2 / s], axis=1)


def _router(xf, Wr, br):
    return pl.pallas_call(
        _router_body,
        out_shape=(
            jax.ShapeDtypeStruct((NT, TOPK), jnp.int32),
            jax.ShapeDtypeStruct((NT, TOPK), jnp.float32),
        ),
    )(xf, Wr, br.reshape(1, NE))


def _plan(idx, gates):
    e_flat = idx.reshape(-1)
    g_flat = gates.reshape(-1)
    counts = jnp.zeros((NE,), jnp.int32).at[e_flat].add(1)
    tiles_e = (counts + TM - 1) // TM
    cum_tiles = jnp.cumsum(tiles_e)
    tile_off_e = cum_tiles - tiles_e
    total_tiles = cum_tiles[-1]
    off_e = jnp.cumsum(counts) - counts
    order = jnp.argsort(e_flat, stable=True).astype(jnp.int32)
    e_sorted = e_flat[order]
    s_idx = jnp.arange(NPAIR, dtype=jnp.int32)
    ps = tile_off_e[e_sorted] * TM + (s_idx - off_e[e_sorted])
    token_src = jnp.zeros((NMAX,), jnp.int32).at[ps].set(order // TOPK)
    gate_sorted = jnp.zeros((NMAX,), jnp.float32).at[ps].set(g_flat[order])
    pos = jnp.zeros((NPAIR,), jnp.int32).at[order].set(ps)
    t_idx = jnp.arange(TILES_MAX, dtype=jnp.int32)
    valid = t_idx < total_tiles
    expert_of_tile = jnp.searchsorted(cum_tiles, t_idx, side="right")
    tile_expert = jnp.where(valid, jnp.clip(expert_of_tile, 0, NE - 1),
                            e_sorted[-1]).astype(jnp.int32)
    row_block = jnp.where(valid, t_idx, total_tiles - 1).astype(jnp.int32)
    return (token_src, gate_sorted, pos[0::2], pos[1::2],
            tile_expert, row_block, valid.astype(jnp.int32))


_DISP_CHUNK = 128
_DISP_PER_W = NMAX // NW


def _dispatch_body(xf_hbm, idx_hbm, out_hbm, idx_v, rows_v, sem):
    wid = lax.axis_index("s") * 2 + lax.axis_index("c")
    for c in range(_DISP_PER_W // _DISP_CHUNK):
        base = wid * _DISP_PER_W + c * _DISP_CHUNK
        pltpu.sync_copy(idx_hbm.at[pl.ds(base, _DISP_CHUNK)], idx_v)
        pltpu.async_copy(xf_hbm.at[idx_v], rows_v, sem).wait()
        pltpu.sync_copy(rows_v, out_hbm.at[pl.ds(base, _DISP_CHUNK)])


def _dispatch(xf, token_src):
    k = functools.partial(
        pl.kernel,
        mesh=plsc.VectorSubcoreMesh(**_SC_MESH),
        out_type=jax.ShapeDtypeStruct((NMAX, DM), jnp.float32),
        scratch_types=[
            pltpu.VMEM((_DISP_CHUNK,), jnp.int32),
            pltpu.VMEM((_DISP_CHUNK, DM), jnp.float32),
            pltpu.SemaphoreType.DMA,
        ],
    )(_dispatch_body)
    return k(xf, token_src)


def _ffn_body(te_ref, rb_ref, va_ref, x_ref, w1_ref, b1_ref, w2_ref, b2_ref,
              g_ref, y_ref):
    t = pl.program_id(0)
    f = pl.program_id(1)

    @pl.when(va_ref[t] == 1)
    def _():
        x = x_ref[...]
        h = jnp.dot(x, w1_ref[0], preferred_element_type=jnp.float32)
        h = h + b1_ref[0]
        h = 0.5 * h * (1.0 + lax.erf(h * 0.7071067811865476))
        yp = jnp.dot(h, w2_ref[0], preferred_element_type=jnp.float32)

        @pl.when(f == 0)
        def _():
            y_ref[...] = yp

        @pl.when(f != 0)
        def _():
            y_ref[...] = y_ref[...] + yp

        @pl.when(f == NFF - 1)
        def _():
            y_ref[...] = (y_ref[...] + b2_ref[0]) * g_ref[...]


def _frozen_f(va_ref, t, f):
    return jnp.where(va_ref[t] == 1, f, NFF - 1)


def _ffn(X_sorted, W1, b1, W2, b2, gate_sorted, tile_expert, row_block, valid):
    grid_spec = pltpu.PrefetchScalarGridSpec(
        num_scalar_prefetch=3,
        grid=(TILES_MAX, NFF),
        in_specs=[
            pl.BlockSpec((TM, DM), lambda t, f, te, rb, va: (rb[t], 0)),
            pl.BlockSpec((1, DM, FF),
                         lambda t, f, te, rb, va: (te[t], 0, _frozen_f(va, t, f))),
            pl.BlockSpec((1, 1, FF),
                         lambda t, f, te, rb, va: (te[t], 0, _frozen_f(va, t, f))),
            pl.BlockSpec((1, FF, DM),
                         lambda t, f, te, rb, va: (te[t], _frozen_f(va, t, f), 0)),
            pl.BlockSpec((1, 1, DM), lambda t, f, te, rb, va: (te[t], 0, 0)),
            pl.BlockSpec((TM, 1), lambda t, f, te, rb, va: (rb[t], 0)),
        ],
        out_specs=pl.BlockSpec((TM, DM), lambda t, f, te, rb, va: (rb[t], 0)),
    )
    return pl.pallas_call(
        _ffn_body,
        grid_spec=grid_spec,
        out_shape=jax.ShapeDtypeStruct((NMAX, DM), jnp.float32),
    )(tile_expert, row_block, valid, X_sorted, W1, b1.reshape(NE, 1, DFF),
      W2, b2.reshape(NE, 1, DM), gate_sorted.reshape(NMAX, 1))


_COMB_PER_W = NT // NW


def _combine_body(y_hbm, p0_hbm, p1_hbm, out_hbm, i0_v, i1_v, r0_v, r1_v, sem):
    wid = lax.axis_index("s") * 2 + lax.axis_index("c")
    base = wid * _COMB_PER_W
    pltpu.sync_copy(p0_hbm.at[pl.ds(base, _COMB_PER_W)], i0_v)
    pltpu.sync_copy(p1_hbm.at[pl.ds(base, _COMB_PER_W)], i1_v)
    pltpu.async_copy(y_hbm.at[i0_v], r0_v, sem).wait()
    pltpu.async_copy(y_hbm.at[i1_v], r1_v, sem).wait()

    def row(rr, _):
        for cc in range(DM // 16):
            sl = pl.ds(cc * 16, 16)
            r0_v[rr, sl] = r0_v[rr, sl] + r1_v[rr, sl]
        return 0

    lax.fori_loop(0, _COMB_PER_W, row, 0)
    pltpu.sync_copy(r0_v, out_hbm.at[pl.ds(base, _COMB_PER_W)])


def _combine(Y, pos0, pos1):
    k = functools.partial(
        pl.kernel,
        mesh=plsc.VectorSubcoreMesh(**_SC_MESH),
        out_type=jax.ShapeDtypeStruct((NT, DM), jnp.float32),
        scratch_types=[
            pltpu.VMEM((_COMB_PER_W,), jnp.int32),
            pltpu.VMEM((_COMB_PER_W,), jnp.int32),
            pltpu.VMEM((_COMB_PER_W, DM), jnp.float32),
            pltpu.VMEM((_COMB_PER_W, DM), jnp.float32),
            pltpu.SemaphoreType.DMA,
        ],
    )(_combine_body)
    return k(Y, pos0, pos1)


def kernel(x, Wr, br, W1, b1, W2, b2):
    B, S, D = x.shape
    xf = x.reshape(-1, D)
    idx, gates = _router(xf, Wr, br)
    token_src, gate_sorted, pos0, pos1, tile_expert, row_block, valid = _plan(
        idx, gates)
    X_sorted = _dispatch(xf, token_src)
    Y = _ffn(X_sorted, W1, b1, W2, b2, gate_sorted, tile_expert, row_block,
             valid)
    out = _combine(Y, pos0, pos1)
    return out.reshape(B, S, D)


# speedup vs baseline: 5.5586x; 1.0025x over previous
import functools

import jax
import jax.numpy as jnp
from jax import lax
from jax.experimental import pallas as pl
from jax.experimental.pallas import tpu as pltpu
from jax.experimental.pallas import tpu_sc as plsc

DM = 768
DFF = 3072
NE = 64
TOPK = 2
NT = 2048
NPAIR = NT * TOPK
TM = 128
TILES_MAX = (NPAIR + NE * TM) // TM
NMAX = TILES_MAX * TM
FF = 3072
NFF = DFF // FF

NW = 32
_SC_MESH = dict(core_axis_name="c", subcore_axis_name="s")


def _router_body(x_ref, wr_ref, br_ref, idx_ref, gate_ref):
    x = x_ref[...]
    logits = jnp.dot(x, wr_ref[...], preferred_element_type=jnp.float32)
    logits = logits + br_ref[...]
    m = jnp.max(logits, axis=1, keepdims=True)
    ex = jnp.exp(logits - m)
    probs = ex / jnp.sum(ex, axis=1, keepdims=True)
    iota = lax.broadcasted_iota(jnp.int32, (NT, NE), 1)
    v1 = jnp.max(probs, axis=1, keepdims=True)
    i1 = jnp.min(jnp.where(probs == v1, iota, NE), axis=1, keepdims=True)
    masked = jnp.where(iota == i1, -1.0, probs)
    v2 = jnp.max(masked, axis=1, keepdims=True)
    i2 = jnp.min(jnp.where(masked == v2, iota, NE), axis=1, keepdims=True)
    s = v1 + v2
    idx_ref[...] = jnp.concatenate([i1, i2], axis=1)
    gate_ref[...] = jnp.concatenate([v1 / s, v2 / s], axis=1)


def _router(xf, Wr, br):
    return pl.pallas_call(
        _router_body,
        out_shape=(
            jax.ShapeDtypeStruct((NT, TOPK), jnp.int32),
            jax.ShapeDtypeStruct((NT, TOPK), jnp.float32),
        ),
    )(xf, Wr, br.reshape(1, NE))


def _plan(idx, gates):
    e_flat = idx.reshape(-1)
    g_flat = gates.reshape(-1)
    counts = jnp.zeros((NE,), jnp.int32).at[e_flat].add(1)
    tiles_e = (counts + TM - 1) // TM
    cum_tiles = jnp.cumsum(tiles_e)
    tile_off_e = cum_tiles - tiles_e
    total_tiles = cum_tiles[-1]
    off_e = jnp.cumsum(counts) - counts
    order = jnp.argsort(e_flat, stable=True).astype(jnp.int32)
    e_sorted = e_flat[order]
    s_idx = jnp.arange(NPAIR, dtype=jnp.int32)
    ps = tile_off_e[e_sorted] * TM + (s_idx - off_e[e_sorted])
    token_src = jnp.zeros((NMAX,), jnp.int32).at[ps].set(order // TOPK)
    gate_sorted = jnp.zeros((NMAX,), jnp.float32).at[ps].set(g_flat[order])
    pos = jnp.zeros((NPAIR,), jnp.int32).at[order].set(ps)
    t_idx = jnp.arange(TILES_MAX, dtype=jnp.int32)
    valid = t_idx < total_tiles
    expert_of_tile = jnp.searchsorted(cum_tiles, t_idx, side="right")
    tile_expert = jnp.where(valid, jnp.clip(expert_of_tile, 0, NE - 1),
                            e_sorted[-1]).astype(jnp.int32)
    row_block = jnp.where(valid, t_idx, total_tiles - 1).astype(jnp.int32)
    return (token_src, gate_sorted, pos[0::2], pos[1::2],
            tile_expert, row_block, valid.astype(jnp.int32))


_DISP_CHUNK = 64
_DISP_PER_W = NMAX // NW
_DISP_NC = _DISP_PER_W // _DISP_CHUNK


def _dispatch_body(xf_hbm, idx_hbm, out_hbm, idx_v, rows0, rows1,
                   sg0, sg1, ss0, ss1):
    wid = lax.axis_index("s") * 2 + lax.axis_index("c")
    base = wid * _DISP_PER_W
    pltpu.sync_copy(idx_hbm.at[pl.ds(base, _DISP_PER_W)], idx_v)
    bufs = (rows0, rows1)
    gsems = (sg0, sg1)
    ssems = (ss0, ss1)
    gathers = {}
    stores = {}

    def gather(c):
        gathers[c] = pltpu.async_copy(
            xf_hbm.at[idx_v.at[pl.ds(c * _DISP_CHUNK, _DISP_CHUNK)]],
            bufs[c % 2], gsems[c % 2])

    gather(0)
    for c in range(_DISP_NC):
        if c + 1 < _DISP_NC:
            if c - 1 >= 0:
                stores[c - 1].wait()
            gather(c + 1)
        gathers[c].wait()
        stores[c] = pltpu.async_copy(
            bufs[c % 2],
            out_hbm.at[pl.ds(base + c * _DISP_CHUNK, _DISP_CHUNK)],
            ssems[c % 2])
    stores[_DISP_NC - 2].wait()
    stores[_DISP_NC - 1].wait()


def _dispatch(xf, token_src):
    k = functools.partial(
        pl.kernel,
        mesh=plsc.VectorSubcoreMesh(**_SC_MESH),
        out_type=jax.ShapeDtypeStruct((NMAX, DM), jnp.float32),
        scratch_types=[
            pltpu.VMEM((_DISP_PER_W,), jnp.int32),
            pltpu.VMEM((_DISP_CHUNK, DM), jnp.float32),
            pltpu.VMEM((_DISP_CHUNK, DM), jnp.float32),
            pltpu.SemaphoreType.DMA,
            pltpu.SemaphoreType.DMA,
            pltpu.SemaphoreType.DMA,
            pltpu.SemaphoreType.DMA,
        ],
    )(_dispatch_body)
    return k(xf, token_src)


def _ffn_body(te_ref, rb_ref, va_ref, x_ref, w1_ref, b1_ref, w2_ref, b2_ref,
              g_ref, y_ref):
    t = pl.program_id(0)
    f = pl.program_id(1)

    @pl.when(va_ref[t] == 1)
    def _():
        x = x_ref[...]
        h = jnp.dot(x, w1_ref[0], preferred_element_type=jnp.float32)
        h = h + b1_ref[0]
        h = 0.5 * h * (1.0 + lax.erf(h * 0.7071067811865476))
        yp = jnp.dot(h, w2_ref[0], preferred_element_type=jnp.float32)

        @pl.when(f == 0)
        def _():
            y_ref[...] = yp

        @pl.when(f != 0)
        def _():
            y_ref[...] = y_ref[...] + yp

        @pl.when(f == NFF - 1)
        def _():
            y_ref[...] = (y_ref[...] + b2_ref[0]) * g_ref[...]


def _frozen_f(va_ref, t, f):
    return jnp.where(va_ref[t] == 1, f, NFF - 1)


def _ffn(X_sorted, W1, b1, W2, b2, gate_sorted, tile_expert, row_block, valid):
    grid_spec = pltpu.PrefetchScalarGridSpec(
        num_scalar_prefetch=3,
        grid=(TILES_MAX, NFF),
        in_specs=[
            pl.BlockSpec((TM, DM), lambda t, f, te, rb, va: (rb[t], 0)),
            pl.BlockSpec((1, DM, FF),
                         lambda t, f, te, rb, va: (te[t], 0, _frozen_f(va, t, f))),
            pl.BlockSpec((1, 1, FF),
                         lambda t, f, te, rb, va: (te[t], 0, _frozen_f(va, t, f))),
            pl.BlockSpec((1, FF, DM),
                         lambda t, f, te, rb, va: (te[t], _frozen_f(va, t, f), 0)),
            pl.BlockSpec((1, 1, DM), lambda t, f, te, rb, va: (te[t], 0, 0)),
            pl.BlockSpec((TM, 1), lambda t, f, te, rb, va: (rb[t], 0)),
        ],
        out_specs=pl.BlockSpec((TM, DM), lambda t, f, te, rb, va: (rb[t], 0)),
    )
    return pl.pallas_call(
        _ffn_body,
        grid_spec=grid_spec,
        out_shape=jax.ShapeDtypeStruct((NMAX, DM), jnp.float32),
    )(tile_expert, row_block, valid, X_sorted, W1, b1.reshape(NE, 1, DFF),
      W2, b2.reshape(NE, 1, DM), gate_sorted.reshape(NMAX, 1))


_COMB_PER_W = NT // NW


def _combine_body(y_hbm, p0_hbm, p1_hbm, out_hbm, i0_v, i1_v, r0_v, r1_v, sem):
    wid = lax.axis_index("s") * 2 + lax.axis_index("c")
    base = wid * _COMB_PER_W
    pltpu.sync_copy(p0_hbm.at[pl.ds(base, _COMB_PER_W)], i0_v)
    pltpu.sync_copy(p1_hbm.at[pl.ds(base, _COMB_PER_W)], i1_v)
    pltpu.async_copy(y_hbm.at[i0_v], r0_v, sem).wait()
    pltpu.async_copy(y_hbm.at[i1_v], r1_v, sem).wait()

    def row(rr, _):
        for cc in range(DM // 16):
            sl = pl.ds(cc * 16, 16)
            r0_v[rr, sl] = r0_v[rr, sl] + r1_v[rr, sl]
        return 0

    lax.fori_loop(0, _COMB_PER_W, row, 0)
    pltpu.sync_copy(r0_v, out_hbm.at[pl.ds(base, _COMB_PER_W)])


def _combine(Y, pos0, pos1):
    k = functools.partial(
        pl.kernel,
        mesh=plsc.VectorSubcoreMesh(**_SC_MESH),
        out_type=jax.ShapeDtypeStruct((NT, DM), jnp.float32),
        scratch_types=[
            pltpu.VMEM((_COMB_PER_W,), jnp.int32),
            pltpu.VMEM((_COMB_PER_W,), jnp.int32),
            pltpu.VMEM((_COMB_PER_W, DM), jnp.float32),
            pltpu.VMEM((_COMB_PER_W, DM), jnp.float32),
            pltpu.SemaphoreType.DMA,
        ],
    )(_combine_body)
    return k(Y, pos0, pos1)


def kernel(x, Wr, br, W1, b1, W2, b2):
    B, S, D = x.shape
    xf = x.reshape(-1, D)
    idx, gates = _router(xf, Wr, br)
    token_src, gate_sorted, pos0, pos1, tile_expert, row_block, valid = _plan(
        idx, gates)
    X_sorted = _dispatch(xf, token_src)
    Y = _ffn(X_sorted, W1, b1, W2, b2, gate_sorted, tile_expert, row_block,
             valid)
    out = _combine(Y, pos0, pos1)
    return out.reshape(B, S, D)


# speedup vs baseline: 8.5644x; 1.5407x over previous
import functools

import jax
import jax.numpy as jnp
from jax import lax
from jax.experimental import pallas as pl
from jax.experimental.pallas import tpu as pltpu
from jax.experimental.pallas import tpu_sc as plsc

DM = 768
DFF = 3072
NE = 64
TOPK = 2
NT = 2048
NPAIR = NT * TOPK
TM = 128
TILES_MAX = (NPAIR + NE * TM) // TM
NMAX = TILES_MAX * TM
FF = 3072
NFF = DFF // FF

NW = 32
_SC_MESH = dict(core_axis_name="c", subcore_axis_name="s")


def _router_body(x_ref, wr_ref, br_ref, idx_ref, gate_ref):
    x = x_ref[...]
    logits = jnp.dot(x, wr_ref[...], preferred_element_type=jnp.float32)
    logits = logits + br_ref[...]
    m = jnp.max(logits, axis=1, keepdims=True)
    ex = jnp.exp(logits - m)
    probs = ex / jnp.sum(ex, axis=1, keepdims=True)
    iota = lax.broadcasted_iota(jnp.int32, (NT, NE), 1)
    v1 = jnp.max(probs, axis=1, keepdims=True)
    i1 = jnp.min(jnp.where(probs == v1, iota, NE), axis=1, keepdims=True)
    masked = jnp.where(iota == i1, -1.0, probs)
    v2 = jnp.max(masked, axis=1, keepdims=True)
    i2 = jnp.min(jnp.where(masked == v2, iota, NE), axis=1, keepdims=True)
    s = v1 + v2
    idx_ref[...] = jnp.concatenate([i1, i2], axis=1)
    gate_ref[...] = jnp.concatenate([v1 / s, v2 / s], axis=1)


def _router(xf, Wr, br):
    return pl.pallas_call(
        _router_body,
        out_shape=(
            jax.ShapeDtypeStruct((NT, TOPK), jnp.int32),
            jax.ShapeDtypeStruct((NT, TOPK), jnp.float32),
        ),
    )(xf, Wr, br.reshape(1, NE))


def _plan(idx, gates):
    e_flat = idx.reshape(-1)
    g_flat = gates.reshape(-1)
    counts = jnp.zeros((NE,), jnp.int32).at[e_flat].add(1)
    tiles_e = (counts + TM - 1) // TM
    cum_tiles = jnp.cumsum(tiles_e)
    tile_off_e = cum_tiles - tiles_e
    total_tiles = cum_tiles[-1]
    off_e = jnp.cumsum(counts) - counts
    order = jnp.argsort(e_flat, stable=True).astype(jnp.int32)
    e_sorted = e_flat[order]
    s_idx = jnp.arange(NPAIR, dtype=jnp.int32)
    ps = tile_off_e[e_sorted] * TM + (s_idx - off_e[e_sorted])
    token_src = (jnp.arange(NMAX, dtype=jnp.int32) % NT).at[ps].set(
        order // TOPK)
    gate_sorted = jnp.zeros((NMAX,), jnp.float32).at[ps].set(g_flat[order])
    pos = jnp.zeros((NPAIR,), jnp.int32).at[order].set(ps)
    t_idx = jnp.arange(TILES_MAX, dtype=jnp.int32)
    valid = t_idx < total_tiles
    expert_of_tile = jnp.searchsorted(cum_tiles, t_idx, side="right")
    tile_expert = jnp.where(valid, jnp.clip(expert_of_tile, 0, NE - 1),
                            e_sorted[-1]).astype(jnp.int32)
    row_block = jnp.where(valid, t_idx, total_tiles - 1).astype(jnp.int32)
    total_rows = jnp.full((16,), total_tiles * TM, jnp.int32)
    return (token_src, gate_sorted, pos[0::2], pos[1::2],
            tile_expert, row_block, valid.astype(jnp.int32), total_rows)


_DISP_CHUNK = 64
_DISP_PER_W = NMAX // NW
_DISP_NC = _DISP_PER_W // _DISP_CHUNK


def _dispatch_body(xf_hbm, idx_hbm, tot_hbm, out_hbm, idx_v, tot_v,
                   rows0, rows1, sg0, sg1, ss0, ss1):
    wid = lax.axis_index("s") * 2 + lax.axis_index("c")
    base = pl.multiple_of(wid * _DISP_PER_W, _DISP_CHUNK)
    bufs = (rows0, rows1)
    gsems = (sg0, sg1)
    ssems = (ss0, ss1)
    gathers = {}
    stores = {}

    def gather(c):
        gathers[c] = pltpu.async_copy(
            xf_hbm.at[idx_v.at[pl.ds(c * _DISP_CHUNK, _DISP_CHUNK)]],
            bufs[c % 2], gsems[c % 2])

    pltpu.sync_copy(idx_hbm.at[pl.ds(base, _DISP_PER_W)], idx_v)
    gather(0)
    for c in range(_DISP_NC):
        if c + 1 < _DISP_NC:
            if c - 1 >= 0:
                stores[c - 1].wait()
            gather(c + 1)
        gathers[c].wait()
        stores[c] = pltpu.async_copy(
            bufs[c % 2],
            out_hbm.at[pl.ds(base + c * _DISP_CHUNK, _DISP_CHUNK)],
            ssems[c % 2])
    stores[_DISP_NC - 2].wait()
    stores[_DISP_NC - 1].wait()


def _dispatch(xf, token_src, total_rows):
    k = functools.partial(
        pl.kernel,
        mesh=plsc.VectorSubcoreMesh(**_SC_MESH),
        out_type=jax.ShapeDtypeStruct((NMAX, DM), jnp.float32),
        scratch_types=[
            pltpu.VMEM((_DISP_PER_W,), jnp.int32),
            pltpu.VMEM((16,), jnp.int32),
            pltpu.VMEM((_DISP_CHUNK, DM), jnp.float32),
            pltpu.VMEM((_DISP_CHUNK, DM), jnp.float32),
            pltpu.SemaphoreType.DMA,
            pltpu.SemaphoreType.DMA,
            pltpu.SemaphoreType.DMA,
            pltpu.SemaphoreType.DMA,
        ],
    )(_dispatch_body)
    return k(xf, token_src, total_rows)


def _ffn_body(te_ref, rb_ref, va_ref, x_ref, w1_ref, b1_ref, w2_ref, b2_ref,
              g_ref, y_ref):
    t = pl.program_id(0)
    f = pl.program_id(1)

    @pl.when(va_ref[t] == 1)
    def _():
        x = x_ref[...]
        h = jnp.dot(x, w1_ref[0], preferred_element_type=jnp.float32)
        h = h + b1_ref[0]
        h = 0.5 * h * (1.0 + lax.erf(h * 0.7071067811865476))
        yp = jnp.dot(h, w2_ref[0], preferred_element_type=jnp.float32)

        @pl.when(f == 0)
        def _():
            y_ref[...] = yp

        @pl.when(f != 0)
        def _():
            y_ref[...] = y_ref[...] + yp

        @pl.when(f == NFF - 1)
        def _():
            y_ref[...] = (y_ref[...] + b2_ref[0]) * g_ref[...]


def _frozen_f(va_ref, t, f):
    return jnp.where(va_ref[t] == 1, f, NFF - 1)


def _ffn(X_sorted, W1, b1, W2, b2, gate_sorted, tile_expert, row_block, valid):
    grid_spec = pltpu.PrefetchScalarGridSpec(
        num_scalar_prefetch=3,
        grid=(TILES_MAX, NFF),
        in_specs=[
            pl.BlockSpec((TM, DM), lambda t, f, te, rb, va: (rb[t], 0)),
            pl.BlockSpec((1, DM, FF),
                         lambda t, f, te, rb, va: (te[t], 0, _frozen_f(va, t, f))),
            pl.BlockSpec((1, 1, FF),
                         lambda t, f, te, rb, va: (te[t], 0, _frozen_f(va, t, f))),
            pl.BlockSpec((1, FF, DM),
                         lambda t, f, te, rb, va: (te[t], _frozen_f(va, t, f), 0)),
            pl.BlockSpec((1, 1, DM), lambda t, f, te, rb, va: (te[t], 0, 0)),
            pl.BlockSpec((TM, 1), lambda t, f, te, rb, va: (rb[t], 0)),
        ],
        out_specs=pl.BlockSpec((TM, DM), lambda t, f, te, rb, va: (rb[t], 0)),
    )
    return pl.pallas_call(
        _ffn_body,
        grid_spec=grid_spec,
        out_shape=jax.ShapeDtypeStruct((NMAX, DM), jnp.float32),
    )(tile_expert, row_block, valid, X_sorted, W1, b1.reshape(NE, 1, DFF),
      W2, b2.reshape(NE, 1, DM), gate_sorted.reshape(NMAX, 1))


_COMB_PER_W = NT // NW


def _combine_body(y_hbm, p0_hbm, p1_hbm, out_hbm, i0_v, i1_v, r0_v, r1_v, sem):
    wid = lax.axis_index("s") * 2 + lax.axis_index("c")
    base = wid * _COMB_PER_W
    pltpu.sync_copy(p0_hbm.at[pl.ds(base, _COMB_PER_W)], i0_v)
    pltpu.sync_copy(p1_hbm.at[pl.ds(base, _COMB_PER_W)], i1_v)
    pltpu.async_copy(y_hbm.at[i0_v], r0_v, sem).wait()
    pltpu.async_copy(y_hbm.at[i1_v], r1_v, sem).wait()

    def row(rr, _):
        for cc in range(DM // 16):
            sl = pl.ds(cc * 16, 16)
            r0_v[rr, sl] = r0_v[rr, sl] + r1_v[rr, sl]
        return 0

    lax.fori_loop(0, _COMB_PER_W, row, 0)
    pltpu.sync_copy(r0_v, out_hbm.at[pl.ds(base, _COMB_PER_W)])


def _combine(Y, pos0, pos1):
    k = functools.partial(
        pl.kernel,
        mesh=plsc.VectorSubcoreMesh(**_SC_MESH),
        out_type=jax.ShapeDtypeStruct((NT, DM), jnp.float32),
        scratch_types=[
            pltpu.VMEM((_COMB_PER_W,), jnp.int32),
            pltpu.VMEM((_COMB_PER_W,), jnp.int32),
            pltpu.VMEM((_COMB_PER_W, DM), jnp.float32),
            pltpu.VMEM((_COMB_PER_W, DM), jnp.float32),
            pltpu.SemaphoreType.DMA,
        ],
    )(_combine_body)
    return k(Y, pos0, pos1)


def kernel(x, Wr, br, W1, b1, W2, b2):
    B, S, D = x.shape
    xf = x.reshape(-1, D)
    idx, gates = _router(xf, Wr, br)
    (token_src, gate_sorted, pos0, pos1, tile_expert, row_block, valid,
     total_rows) = _plan(idx, gates)
    X_sorted = _dispatch(xf, token_src, total_rows)
    Y = _ffn(X_sorted, W1, b1, W2, b2, gate_sorted, tile_expert, row_block,
             valid)
    out = _combine(Y, pos0, pos1)
    return out.reshape(B, S, D)


# speedup vs baseline: 10.5850x; 1.2359x over previous
import functools

import jax
import jax.numpy as jnp
from jax import lax
from jax.experimental import pallas as pl
from jax.experimental.pallas import tpu as pltpu
from jax.experimental.pallas import tpu_sc as plsc

DM = 768
DFF = 3072
NE = 64
TOPK = 2
NT = 2048
NPAIR = NT * TOPK
TM = 128
TILES_MAX = (NPAIR + NE * TM) // TM
NMAX = TILES_MAX * TM
FF = 3072
NFF = DFF // FF

NW = 32
_SC_MESH = dict(core_axis_name="c", subcore_axis_name="s")


def _router_body(x_ref, wr_ref, br_ref, idx_ref, g0_ref, g1_ref, rank_ref, cnt_ref):
    x = x_ref[...]
    logits = jnp.dot(x, wr_ref[...], preferred_element_type=jnp.float32)
    logits = logits + br_ref[...]
    m = jnp.max(logits, axis=1, keepdims=True)
    ex = jnp.exp(logits - m)
    probs = ex / jnp.sum(ex, axis=1, keepdims=True)
    iota = lax.broadcasted_iota(jnp.int32, (NT, NE), 1)
    v1 = jnp.max(probs, axis=1, keepdims=True)
    i1 = jnp.min(jnp.where(probs == v1, iota, NE), axis=1, keepdims=True)
    masked = jnp.where(iota == i1, -1.0, probs)
    v2 = jnp.max(masked, axis=1, keepdims=True)
    i2 = jnp.min(jnp.where(masked == v2, iota, NE), axis=1, keepdims=True)
    s = v1 + v2
    idx_ref[...] = jnp.concatenate([i1, i2], axis=1)
    g0_ref[...] = jnp.broadcast_to(v1 / s, (NT, 16))
    g1_ref[...] = jnp.broadcast_to(v2 / s, (NT, 16))

    e_cat = jnp.concatenate([i1, i2], axis=0)
    piota = lax.broadcasted_iota(jnp.int32, (NPAIR, NE), 1)
    oh = (e_cat == piota).astype(jnp.int32)
    c = oh
    k = 1
    while k < NPAIR:
        top = jnp.zeros((k, NE), jnp.int32)
        c = c + jnp.concatenate([top, c[:NPAIR - k]], axis=0)
        k *= 2
    rank_ref[...] = jnp.sum(oh * c, axis=1, keepdims=True) - 1
    cnt_ref[...] = jnp.sum(oh, axis=0, keepdims=True)


def _router(xf, Wr, br):
    return pl.pallas_call(
        _router_body,
        out_shape=(
            jax.ShapeDtypeStruct((NT, TOPK), jnp.int32),
            jax.ShapeDtypeStruct((NT, 16), jnp.float32),
            jax.ShapeDtypeStruct((NT, 16), jnp.float32),
            jax.ShapeDtypeStruct((NPAIR, 1), jnp.int32),
            jax.ShapeDtypeStruct((1, NE), jnp.int32),
        ),
    )(xf, Wr, br.reshape(1, NE))


def _plan(idx, rank, counts):
    counts = counts.reshape(NE)
    tiles_e = (counts + TM - 1) // TM
    cum_tiles = jnp.cumsum(tiles_e)
    tile_off_e = cum_tiles - tiles_e
    total_tiles = cum_tiles[-1]
    e_cat = jnp.concatenate([idx[:, 0], idx[:, 1]])
    ps = tile_off_e[e_cat] * TM + rank.reshape(-1)
    t_idx = jnp.arange(TILES_MAX, dtype=jnp.int32)
    valid = t_idx < total_tiles
    expert_of_tile = jnp.searchsorted(cum_tiles, t_idx, side="right")
    e_last = jnp.max(jnp.where(counts > 0, jnp.arange(NE, dtype=jnp.int32),
                               -1))
    tile_expert = jnp.where(valid, jnp.clip(expert_of_tile, 0, NE - 1),
                            e_last).astype(jnp.int32)
    row_block = jnp.where(valid, t_idx, total_tiles - 1).astype(jnp.int32)
    return ps, ps[:NT], ps[NT:], tile_expert, row_block, valid.astype(jnp.int32)


_DISP_PER_W = NPAIR // NW


def _dispatch_body(xf_hbm, ps_hbm, out_hbm, idx_v, rows_v, sem):
    wid = lax.axis_index("s") * 2 + lax.axis_index("c")
    pbase = pl.multiple_of(wid * _DISP_PER_W, _DISP_PER_W)
    tbase = pl.multiple_of(jnp.remainder(wid, NW // 2) * _DISP_PER_W,
                           _DISP_PER_W)
    pltpu.sync_copy(ps_hbm.at[pl.ds(pbase, _DISP_PER_W)], idx_v)
    pltpu.sync_copy(xf_hbm.at[pl.ds(tbase, _DISP_PER_W)], rows_v)
    pltpu.async_copy(rows_v, out_hbm.at[idx_v], sem).wait()


def _dispatch(xf, ps):
    k = functools.partial(
        pl.kernel,
        mesh=plsc.VectorSubcoreMesh(**_SC_MESH),
        out_type=jax.ShapeDtypeStruct((NMAX, DM), jnp.float32),
        scratch_types=[
            pltpu.VMEM((_DISP_PER_W,), jnp.int32),
            pltpu.VMEM((_DISP_PER_W, DM), jnp.float32),
            pltpu.SemaphoreType.DMA,
        ],
    )(_dispatch_body)
    return k(xf, ps)


def _ffn_body(te_ref, rb_ref, va_ref, x_ref, w1_ref, b1_ref, w2_ref, b2_ref,
              y_ref):
    t = pl.program_id(0)
    f = pl.program_id(1)

    @pl.when(va_ref[t] == 1)
    def _():
        x = x_ref[...]
        h = jnp.dot(x, w1_ref[0], preferred_element_type=jnp.float32)
        h = h + b1_ref[0]
        h = 0.5 * h * (1.0 + lax.erf(h * 0.7071067811865476))
        yp = jnp.dot(h, w2_ref[0], preferred_element_type=jnp.float32)

        @pl.when(f == 0)
        def _():
            y_ref[...] = yp

        @pl.when(f != 0)
        def _():
            y_ref[...] = y_ref[...] + yp

        @pl.when(f == NFF - 1)
        def _():
            y_ref[...] = y_ref[...] + b2_ref[0]


def _frozen_f(va_ref, t, f):
    return jnp.where(va_ref[t] == 1, f, NFF - 1)


def _ffn(X_sorted, W1, b1, W2, b2, tile_expert, row_block, valid):
    grid_spec = pltpu.PrefetchScalarGridSpec(
        num_scalar_prefetch=3,
        grid=(TILES_MAX, NFF),
        in_specs=[
            pl.BlockSpec((TM, DM), lambda t, f, te, rb, va: (rb[t], 0)),
            pl.BlockSpec((1, DM, FF),
                         lambda t, f, te, rb, va: (te[t], 0, _frozen_f(va, t, f))),
            pl.BlockSpec((1, 1, FF),
                         lambda t, f, te, rb, va: (te[t], 0, _frozen_f(va, t, f))),
            pl.BlockSpec((1, FF, DM),
                         lambda t, f, te, rb, va: (te[t], _frozen_f(va, t, f), 0)),
            pl.BlockSpec((1, 1, DM), lambda t, f, te, rb, va: (te[t], 0, 0)),
        ],
        out_specs=pl.BlockSpec((TM, DM), lambda t, f, te, rb, va: (rb[t], 0)),
    )
    return pl.pallas_call(
        _ffn_body,
        grid_spec=grid_spec,
        out_shape=jax.ShapeDtypeStruct((NMAX, DM), jnp.float32),
    )(tile_expert, row_block, valid, X_sorted, W1, b1.reshape(NE, 1, DFF),
      W2, b2.reshape(NE, 1, DM))


_COMB_PER_W = NT // NW


def _combine_body(y_hbm, p0_hbm, p1_hbm, g0_hbm, g1_hbm, out_hbm,
                  i0_v, i1_v, g0_v, g1_v, r0_v, r1_v, sem):
    wid = lax.axis_index("s") * 2 + lax.axis_index("c")
    base = pl.multiple_of(wid * _COMB_PER_W, _COMB_PER_W)
    pltpu.sync_copy(p0_hbm.at[pl.ds(base, _COMB_PER_W)], i0_v)
    pltpu.sync_copy(p1_hbm.at[pl.ds(base, _COMB_PER_W)], i1_v)
    pltpu.sync_copy(g0_hbm.at[pl.ds(base, _COMB_PER_W)], g0_v)
    pltpu.sync_copy(g1_hbm.at[pl.ds(base, _COMB_PER_W)], g1_v)
    pltpu.async_copy(y_hbm.at[i0_v], r0_v, sem).wait()
    pltpu.async_copy(y_hbm.at[i1_v], r1_v, sem).wait()

    def row(rr, _):
        g0b = g0_v[rr, :]
        g1b = g1_v[rr, :]
        for cc in range(DM // 16):
            sl = pl.ds(cc * 16, 16)
            r0_v[rr, sl] = r0_v[rr, sl] * g0b + r1_v[rr, sl] * g1b
        return 0

    lax.fori_loop(0, _COMB_PER_W, row, 0)
    pltpu.sync_copy(r0_v, out_hbm.at[pl.ds(base, _COMB_PER_W)])


def _combine(Y, pos0, pos1, g0, g1):
    k = functools.partial(
        pl.kernel,
        mesh=plsc.VectorSubcoreMesh(**_SC_MESH),
        out_type=jax.ShapeDtypeStruct((NT, DM), jnp.float32),
        scratch_types=[
            pltpu.VMEM((_COMB_PER_W,), jnp.int32),
            pltpu.VMEM((_COMB_PER_W,), jnp.int32),
            pltpu.VMEM((_COMB_PER_W, 16), jnp.float32),
            pltpu.VMEM((_COMB_PER_W, 16), jnp.float32),
            pltpu.VMEM((_COMB_PER_W, DM), jnp.float32),
            pltpu.VMEM((_COMB_PER_W, DM), jnp.float32),
            pltpu.SemaphoreType.DMA,
        ],
    )(_combine_body)
    return k(Y, pos0, pos1, g0, g1)


def kernel(x, Wr, br, W1, b1, W2, b2):
    B, S, D = x.shape
    xf = x.reshape(-1, D)
    idx, g0x, g1x, rank, counts = _router(xf, Wr, br)
    ps, pos0, pos1, tile_expert, row_block, valid = _plan(idx, rank, counts)
    X_sorted = _dispatch(xf, ps)
    Y = _ffn(X_sorted, W1, b1, W2, b2, tile_expert, row_block, valid)
    out = _combine(Y, pos0, pos1, g0x, g1x)
    return out.reshape(B, S, D)


# speedup vs baseline: 11.7937x; 1.1142x over previous
import functools

import jax
import jax.numpy as jnp
from jax import lax
from jax.experimental import pallas as pl
from jax.experimental.pallas import tpu as pltpu
from jax.experimental.pallas import tpu_sc as plsc

DM = 768
DFF = 3072
NE = 64
TOPK = 2
NT = 2048
NPAIR = NT * TOPK
TM = 128
TILES_MAX = (NPAIR + NE * TM) // TM
NMAX = TILES_MAX * TM
FF = 3072
NFF = DFF // FF

NW = 32
_SC_MESH = dict(core_axis_name="c", subcore_axis_name="s")


def _router_body(x_ref, wr_ref, br_ref, g0_ref, g1_ref, ps_ref,
                 te_ref, rb_ref, va_ref):
    x = x_ref[...]
    logits = jnp.dot(x, wr_ref[...], preferred_element_type=jnp.float32)
    logits = logits + br_ref[...]
    m = jnp.max(logits, axis=1, keepdims=True)
    ex = jnp.exp(logits - m)
    probs = ex / jnp.sum(ex, axis=1, keepdims=True)
    iota = lax.broadcasted_iota(jnp.int32, (NT, NE), 1)
    v1 = jnp.max(probs, axis=1, keepdims=True)
    i1 = jnp.min(jnp.where(probs == v1, iota, NE), axis=1, keepdims=True)
    masked = jnp.where(iota == i1, -1.0, probs)
    v2 = jnp.max(masked, axis=1, keepdims=True)
    i2 = jnp.min(jnp.where(masked == v2, iota, NE), axis=1, keepdims=True)
    s = v1 + v2
    g0_ref[...] = jnp.broadcast_to(v1 / s, (NT, 16))
    g1_ref[...] = jnp.broadcast_to(v2 / s, (NT, 16))

    e_cat = jnp.concatenate([i1, i2], axis=0)
    piota = lax.broadcasted_iota(jnp.int32, (NPAIR, NE), 1)
    oh = (e_cat == piota).astype(jnp.int32)
    c = oh
    k = 1
    while k < NPAIR:
        top = jnp.zeros((k, NE), jnp.int32)
        c = c + jnp.concatenate([top, c[:NPAIR - k]], axis=0)
        k *= 2
    rank = jnp.sum(oh * c, axis=1, keepdims=True) - 1
    counts = jnp.sum(oh, axis=0, keepdims=True)

    tiles_e = lax.shift_right_logical(counts + (TM - 1), TM.bit_length() - 1)
    cum = tiles_e
    k = 1
    while k < NE:
        left = jnp.zeros((1, k), jnp.int32)
        cum = cum + jnp.concatenate([left, cum[:, :NE - k]], axis=1)
        k *= 2
    tile_off = cum - tiles_e
    total = cum[:, NE - 1:]
    ps_ref[...] = TM * jnp.sum(oh * tile_off, axis=1, keepdims=True) + rank

    t_col = lax.broadcasted_iota(jnp.int32, (TILES_MAX, 1), 0)
    t_mat = lax.broadcasted_iota(jnp.int32, (TILES_MAX, NE), 0)
    expert_of_tile = jnp.sum((jnp.broadcast_to(cum, (TILES_MAX, NE)) <=
                              t_mat).astype(jnp.int32), axis=1, keepdims=True)
    eiota = lax.broadcasted_iota(jnp.int32, (1, NE), 1)
    e_last = jnp.max(jnp.where(counts > 0, eiota, -1), axis=1, keepdims=True)
    valid = t_col < total
    te_ref[...] = jnp.where(valid, jnp.clip(expert_of_tile, 0, NE - 1), e_last)
    rb_ref[...] = jnp.where(valid, t_col, total - 1)
    va_ref[...] = valid.astype(jnp.int32)


def _router(xf, Wr, br):
    return pl.pallas_call(
        _router_body,
        out_shape=(
            jax.ShapeDtypeStruct((NT, 16), jnp.float32),
            jax.ShapeDtypeStruct((NT, 16), jnp.float32),
            jax.ShapeDtypeStruct((NPAIR, 1), jnp.int32),
            jax.ShapeDtypeStruct((TILES_MAX, 1), jnp.int32),
            jax.ShapeDtypeStruct((TILES_MAX, 1), jnp.int32),
            jax.ShapeDtypeStruct((TILES_MAX, 1), jnp.int32),
        ),
    )(xf, Wr, br.reshape(1, NE))


_DISP_PER_W = NPAIR // NW


def _dispatch_body(xf_hbm, ps_hbm, out_hbm, idx_v, rows_v, sem):
    wid = lax.axis_index("s") * 2 + lax.axis_index("c")
    pbase = pl.multiple_of(wid * _DISP_PER_W, _DISP_PER_W)
    tbase = pl.multiple_of(jnp.remainder(wid, NW // 2) * _DISP_PER_W,
                           _DISP_PER_W)
    pltpu.sync_copy(ps_hbm.at[pl.ds(pbase, _DISP_PER_W)], idx_v)
    pltpu.sync_copy(xf_hbm.at[pl.ds(tbase, _DISP_PER_W)], rows_v)
    pltpu.async_copy(rows_v, out_hbm.at[idx_v], sem).wait()


def _dispatch(xf, ps):
    k = functools.partial(
        pl.kernel,
        mesh=plsc.VectorSubcoreMesh(**_SC_MESH),
        out_type=jax.ShapeDtypeStruct((NMAX, DM), jnp.float32),
        scratch_types=[
            pltpu.VMEM((_DISP_PER_W,), jnp.int32),
            pltpu.VMEM((_DISP_PER_W, DM), jnp.float32),
            pltpu.SemaphoreType.DMA,
        ],
    )(_dispatch_body)
    return k(xf, ps)


def _ffn_body(te_ref, rb_ref, va_ref, x_ref, w1_ref, b1_ref, w2_ref, b2_ref,
              y_ref):
    t = pl.program_id(0)
    f = pl.program_id(1)

    @pl.when(va_ref[t] == 1)
    def _():
        x = x_ref[...]
        h = jnp.dot(x, w1_ref[0], preferred_element_type=jnp.float32)
        h = h + b1_ref[0]
        h = 0.5 * h * (1.0 + lax.erf(h * 0.7071067811865476))
        yp = jnp.dot(h, w2_ref[0], preferred_element_type=jnp.float32)

        @pl.when(f == 0)
        def _():
            y_ref[...] = yp

        @pl.when(f != 0)
        def _():
            y_ref[...] = y_ref[...] + yp

        @pl.when(f == NFF - 1)
        def _():
            y_ref[...] = y_ref[...] + b2_ref[0]


def _frozen_f(va_ref, t, f):
    return jnp.where(va_ref[t] == 1, f, NFF - 1)


def _ffn(X_sorted, W1, b1, W2, b2, tile_expert, row_block, valid):
    grid_spec = pltpu.PrefetchScalarGridSpec(
        num_scalar_prefetch=3,
        grid=(TILES_MAX, NFF),
        in_specs=[
            pl.BlockSpec((TM, DM), lambda t, f, te, rb, va: (rb[t], 0)),
            pl.BlockSpec((1, DM, FF),
                         lambda t, f, te, rb, va: (te[t], 0, _frozen_f(va, t, f))),
            pl.BlockSpec((1, 1, FF),
                         lambda t, f, te, rb, va: (te[t], 0, _frozen_f(va, t, f))),
            pl.BlockSpec((1, FF, DM),
                         lambda t, f, te, rb, va: (te[t], _frozen_f(va, t, f), 0)),
            pl.BlockSpec((1, 1, DM), lambda t, f, te, rb, va: (te[t], 0, 0)),
        ],
        out_specs=pl.BlockSpec((TM, DM), lambda t, f, te, rb, va: (rb[t], 0)),
    )
    return pl.pallas_call(
        _ffn_body,
        grid_spec=grid_spec,
        out_shape=jax.ShapeDtypeStruct((NMAX, DM), jnp.float32),
    )(tile_expert, row_block, valid, X_sorted, W1, b1.reshape(NE, 1, DFF),
      W2, b2.reshape(NE, 1, DM))


_COMB_PER_W = NT // NW


def _combine_body(y_hbm, p0_hbm, p1_hbm, g0_hbm, g1_hbm, out_hbm,
                  i0_v, i1_v, g0_v, g1_v, r0_v, r1_v, sem):
    wid = lax.axis_index("s") * 2 + lax.axis_index("c")
    base = pl.multiple_of(wid * _COMB_PER_W, _COMB_PER_W)
    pltpu.sync_copy(p0_hbm.at[pl.ds(base, _COMB_PER_W)], i0_v)
    pltpu.sync_copy(p1_hbm.at[pl.ds(base, _COMB_PER_W)], i1_v)
    pltpu.sync_copy(g0_hbm.at[pl.ds(base, _COMB_PER_W)], g0_v)
    pltpu.sync_copy(g1_hbm.at[pl.ds(base, _COMB_PER_W)], g1_v)
    pltpu.async_copy(y_hbm.at[i0_v], r0_v, sem).wait()
    pltpu.async_copy(y_hbm.at[i1_v], r1_v, sem).wait()

    def row(rr, _):
        g0b = g0_v[rr, :]
        g1b = g1_v[rr, :]
        for cc in range(DM // 16):
            sl = pl.ds(cc * 16, 16)
            r0_v[rr, sl] = r0_v[rr, sl] * g0b + r1_v[rr, sl] * g1b
        return 0

    lax.fori_loop(0, _COMB_PER_W, row, 0)
    pltpu.sync_copy(r0_v, out_hbm.at[pl.ds(base, _COMB_PER_W)])


def _combine(Y, pos0, pos1, g0, g1):
    k = functools.partial(
        pl.kernel,
        mesh=plsc.VectorSubcoreMesh(**_SC_MESH),
        out_type=jax.ShapeDtypeStruct((NT, DM), jnp.float32),
        scratch_types=[
            pltpu.VMEM((_COMB_PER_W,), jnp.int32),
            pltpu.VMEM((_COMB_PER_W,), jnp.int32),
            pltpu.VMEM((_COMB_PER_W, 16), jnp.float32),
            pltpu.VMEM((_COMB_PER_W, 16), jnp.float32),
            pltpu.VMEM((_COMB_PER_W, DM), jnp.float32),
            pltpu.VMEM((_COMB_PER_W, DM), jnp.float32),
            pltpu.SemaphoreType.DMA,
        ],
    )(_combine_body)
    return k(Y, pos0, pos1, g0, g1)


def kernel(x, Wr, br, W1, b1, W2, b2):
    B, S, D = x.shape
    xf = x.reshape(-1, D)
    g0x, g1x, ps2, te2, rb2, va2 = _router(xf, Wr, br)
    ps = ps2.reshape(NPAIR)
    X_sorted = _dispatch(xf, ps)
    Y = _ffn(X_sorted, W1, b1, W2, b2, te2.reshape(TILES_MAX),
             rb2.reshape(TILES_MAX), va2.reshape(TILES_MAX))
    out = _combine(Y, ps[:NT], ps[NT:], g0x, g1x)
    return out.reshape(B, S, D)
